# baseline (speedup 1.0000x reference)
import functools

import jax
import jax.numpy as jnp
from jax import lax
from jax.experimental import pallas as pl
from jax.experimental.pallas import tpu as pltpu
from jax.experimental.pallas import tpu_sc as plsc

_NW = 32
_TILE = 128
_CT_PER_CHUNK = 15
_CHUNK = _TILE * _CT_PER_CHUNK
_CHUNKS_PER_W = 26
_UNROLL = 8


def _sc_call(num_atoms, atoms_per_b, num_edges, nelem):
    vecs = _CHUNK // 16
    main_tiles = _NW * _CHUNKS_PER_W * _CT_PER_CHUNK
    tail_tiles = num_edges // _TILE - main_tiles
    mesh = plsc.VectorSubcoreMesh(core_axis_name="c", subcore_axis_name="s")

    @functools.partial(
        pl.kernel,
        out_type=jax.ShapeDtypeStruct((_NW, 256), jnp.float32),
        mesh=mesh,
        scratch_types=[
            pltpu.VMEM((num_atoms,), jnp.int32),
            pltpu.VMEM((num_atoms,), jnp.int32),
            pltpu.VMEM((nelem * nelem,), jnp.int32),
            pltpu.VMEM((3, _CHUNK), jnp.int32),
            pltpu.VMEM((3, _CHUNK), jnp.int32),
            pltpu.VMEM((256,), jnp.float32),
            pltpu.SemaphoreType.DMA,
            pltpu.SemaphoreType.DMA,
            pltpu.SemaphoreType.DMA,
        ],
        compiler_params=pltpu.CompilerParams(needs_layout_passes=False),
    )
    def body(tbl0_h, tbl1_h, kr_h, adj_h, out_h,
             tbl0, tbl1, krv, ab0, ab1, acc, semt, sem0, sem1):
        wid = lax.axis_index("s") * 2 + lax.axis_index("c")
        base_w = wid * (_CHUNKS_PER_W * _CHUNK)

        def chunk_src(c):
            return adj_h.at[:, pl.ds(base_w + c * _CHUNK, _CHUNK)]

        pltpu.async_copy(chunk_src(0), ab0, sem0)
        dt0 = pltpu.async_copy(tbl0_h, tbl0, semt)
        dt1 = pltpu.async_copy(tbl1_h, tbl1, semt)
        dt2 = pltpu.async_copy(kr_h, krv, semt)
        for r in range(16):
            acc[pl.ds(r * 16, 16)] = jnp.zeros((16,), jnp.float32)
        lane = lax.iota(jnp.int32, 16)
        mh = jnp.int32(-65536)
        dt0.wait()
        dt1.wait()
        dt2.wait()

        def compute(ab, nvec):
            @plsc.parallel_loop(0, nvec, 1, unroll=_UNROLL)
            def vec_body(v):
                o = v * 16
                nn = ab[0, pl.ds(o, 16)]
                ii = ab[1, pl.ds(o, 16)]
                jj = ab[2, pl.ds(o, 16)]
                gb = nn * atoms_per_b
                gi = gb + ii
                gj = gb + jj
                w0i = plsc.load_gather(tbl0, [gi])
                w1i = plsc.load_gather(tbl1, [gi])
                w0j = plsc.load_gather(tbl0, [gj])
                w1j = plsc.load_gather(tbl1, [gj])
                b0i = plsc.bitcast(w0i, jnp.bfloat16)
                b0j = plsc.bitcast(w0j, jnp.bfloat16)
                dxy = b0j - b0i
                sq = dxy * dxy
                s_a, s_b = plsc.unpack(sq, format=plsc.PackFormat.INTERLEAVED)
                zi = plsc.bitcast(w1i & jnp.int32(-64), jnp.float32)
                zj = plsc.bitcast(w1j & jnp.int32(-64), jnp.float32)
                code = (w1i & 56) | (w1j & 7)
                krc = plsc.load_gather(krv, [code])
                kk = plsc.bitcast(krc & mh, jnp.float32)
                rr = plsc.bitcast(krc << 16, jnp.float32)
                dz = zj - zi
                sod = s_a + s_b + dz * dz
                yh = plsc.bitcast(
                    jnp.int32(0x5F3759DF) - (plsc.bitcast(sod, jnp.int32) >> 1),
                    jnp.float32)
                half = 0.5 * sod
                yh = yh * (1.5 - half * yh * yh)
                yh = yh * (1.5 - half * yh * yh)
                dis = sod * yh
                dd = jnp.minimum(dis - rr, jnp.float32(0.0))
                eng = kk * dd * dd
                plsc.addupdate_scatter(acc, [(nn << 4) + lane], eng)

        def pair_body(g, carry):
            c0 = 2 * g
            pltpu.async_copy(chunk_src(c0 + 1), ab1, sem1)
            pltpu.make_async_copy(chunk_src(c0), ab0, sem0).wait()
            compute(ab0, vecs)

            @pl.when(c0 + 2 < _CHUNKS_PER_W)
            def _():
                pltpu.async_copy(chunk_src(c0 + 2), ab0, sem0)

            pltpu.make_async_copy(chunk_src(c0 + 1), ab1, sem1).wait()
            compute(ab1, vecs)
            return carry

        lax.fori_loop(0, _CHUNKS_PER_W // 2, pair_body, 0, unroll=False)

        @pl.when(wid < tail_tiles)
        def _():
            tail_src = adj_h.at[:, pl.ds((main_tiles + wid) * _TILE, _TILE)]
            tail_dst = ab0.at[:, pl.ds(0, _TILE)]
            pltpu.async_copy(tail_src, tail_dst, sem0)
            pltpu.make_async_copy(tail_src, tail_dst, sem0).wait()
            compute(ab0, _TILE // 16)

        pltpu.sync_copy(acc, out_h.at[wid])

    return body


def kernel(pos, elm, adj, k, radius):
    B, A, _ = pos.shape
    E = adj.shape[1]
    nelem = k.shape[0]

    pf = pos.reshape(B * A, 3)

    def to16(v):
        return lax.bitcast_convert_type(
            v.astype(jnp.bfloat16), jnp.uint16).astype(jnp.uint32)

    tbl0 = lax.bitcast_convert_type((to16(pf[:, 0]) << 16) | to16(pf[:, 1]),
                                    jnp.int32)
    zb = lax.bitcast_convert_type(pf[:, 2], jnp.uint32)
    tbl1 = lax.bitcast_convert_type(
        (zb & jnp.uint32(0xFFFFFFC0))
        | (elm.reshape(-1).astype(jnp.uint32) * 9),
        jnp.int32)
    kk_tbl = (k[:, None] + k[None, :]).reshape(-1)
    rr_tbl = (radius[:, None] + radius[None, :]).reshape(-1)
    kr_tbl = lax.bitcast_convert_type((to16(kk_tbl) << 16) | to16(rr_tbl),
                                      jnp.int32)

    call = _sc_call(B * A, A, E, nelem)
    partial = call(tbl0, tbl1, kr_tbl, adj.astype(jnp.int32))
    return partial.reshape(_NW, 16, 16).sum(axis=(0, 2))


# speedup vs baseline: 1.0394x; 1.0394x over previous
import functools

import jax
import jax.numpy as jnp
from jax import lax
from jax.experimental import pallas as pl
from jax.experimental.pallas import tpu as pltpu
from jax.experimental.pallas import tpu_sc as plsc

_NW = 32
_TILE = 128
_CT_PER_CHUNK = 15
_CHUNK = _TILE * _CT_PER_CHUNK
_CHUNKS_PER_W = 26
_UNROLL = 8


def _sc_call(num_atoms, atoms_per_b, num_edges, nelem):
    vecs = _CHUNK // 16
    main_tiles = _NW * _CHUNKS_PER_W * _CT_PER_CHUNK
    tail_tiles = num_edges // _TILE - main_tiles
    mesh = plsc.VectorSubcoreMesh(core_axis_name="c", subcore_axis_name="s")

    @functools.partial(
        pl.kernel,
        out_type=jax.ShapeDtypeStruct((_NW, 256), jnp.float32),
        mesh=mesh,
        scratch_types=[
            pltpu.VMEM((num_atoms,), jnp.int32),
            pltpu.VMEM((num_atoms,), jnp.int32),
            pltpu.VMEM((nelem * nelem,), jnp.int32),
            pltpu.VMEM((3, _CHUNK), jnp.int32),
            pltpu.VMEM((3, _CHUNK), jnp.int32),
            pltpu.VMEM((256,), jnp.float32),
            pltpu.SemaphoreType.DMA,
            pltpu.SemaphoreType.DMA,
            pltpu.SemaphoreType.DMA,
        ],
        compiler_params=pltpu.CompilerParams(needs_layout_passes=False),
    )
    def body(tbl0_h, tbl1_h, kr_h, adj_h, out_h,
             tbl0, tbl1, krv, ab0, ab1, acc, semt, sem0, sem1):
        wid = lax.axis_index("s") * 2 + lax.axis_index("c")
        base_w = wid * (_CHUNKS_PER_W * _CHUNK)

        def chunk_src(c):
            return adj_h.at[:, pl.ds(base_w + c * _CHUNK, _CHUNK)]

        pltpu.async_copy(chunk_src(0), ab0, sem0)
        dt0 = pltpu.async_copy(tbl0_h, tbl0, semt)
        dt1 = pltpu.async_copy(tbl1_h, tbl1, semt)
        dt2 = pltpu.async_copy(kr_h, krv, semt)
        for r in range(16):
            acc[pl.ds(r * 16, 16)] = jnp.zeros((16,), jnp.float32)
        lane = lax.iota(jnp.int32, 16)
        mh = jnp.int32(-65536)
        dt0.wait()
        dt1.wait()
        dt2.wait()

        def compute(ab, nvec):
            @plsc.parallel_loop(0, nvec, 1, unroll=_UNROLL)
            def vec_body(v):
                o = v * 16
                nn = ab[0, pl.ds(o, 16)]
                ii = ab[1, pl.ds(o, 16)]
                jj = ab[2, pl.ds(o, 16)]
                gb = nn * atoms_per_b
                gi = gb + ii
                gj = gb + jj
                w0i = plsc.load_gather(tbl0, [gi])
                w1i = plsc.load_gather(tbl1, [gi])
                w0j = plsc.load_gather(tbl0, [gj])
                w1j = plsc.load_gather(tbl1, [gj])
                xi = plsc.bitcast(w0i & mh, jnp.float32)
                yi = plsc.bitcast(w0i << 16, jnp.float32)
                xj = plsc.bitcast(w0j & mh, jnp.float32)
                yj = plsc.bitcast(w0j << 16, jnp.float32)
                zi = plsc.bitcast(w1i & jnp.int32(-64), jnp.float32)
                zj = plsc.bitcast(w1j & jnp.int32(-64), jnp.float32)
                code = (w1i & 56) | (w1j & 7)
                krc = plsc.load_gather(krv, [code])
                kk = plsc.bitcast(krc & mh, jnp.float32)
                rr = plsc.bitcast(krc << 16, jnp.float32)
                dx = xj - xi
                dy = yj - yi
                dz = zj - zi
                sod = dx * dx + dy * dy + dz * dz
                yh = plsc.bitcast(
                    jnp.int32(0x5F3759DF) - (plsc.bitcast(sod, jnp.int32) >> 1),
                    jnp.float32)
                half = 0.5 * sod
                yh = yh * (1.5 - half * yh * yh)
                yh = yh * (1.5 - half * yh * yh)
                dis = sod * yh
                dd = jnp.minimum(dis - rr, jnp.float32(0.0))
                eng = kk * dd * dd
                plsc.addupdate_scatter(acc, [(nn << 4) + lane], eng)

        def pair_body(g, carry):
            c0 = 2 * g
            pltpu.async_copy(chunk_src(c0 + 1), ab1, sem1)
            pltpu.make_async_copy(chunk_src(c0), ab0, sem0).wait()
            compute(ab0, vecs)

            @pl.when(c0 + 2 < _CHUNKS_PER_W)
            def _():
                pltpu.async_copy(chunk_src(c0 + 2), ab0, sem0)

            pltpu.make_async_copy(chunk_src(c0 + 1), ab1, sem1).wait()
            compute(ab1, vecs)
            return carry

        lax.fori_loop(0, _CHUNKS_PER_W // 2, pair_body, 0, unroll=False)

        @pl.when(wid < tail_tiles)
        def _():
            tail_src = adj_h.at[:, pl.ds((main_tiles + wid) * _TILE, _TILE)]
            tail_dst = ab0.at[:, pl.ds(0, _TILE)]
            pltpu.async_copy(tail_src, tail_dst, sem0)
            pltpu.make_async_copy(tail_src, tail_dst, sem0).wait()
            compute(ab0, _TILE // 16)

        pltpu.sync_copy(acc, out_h.at[wid])

    return body


def kernel(pos, elm, adj, k, radius):
    B, A, _ = pos.shape
    E = adj.shape[1]
    nelem = k.shape[0]

    pf = pos.reshape(B * A, 3)

    def to16(v):
        return lax.bitcast_convert_type(
            v.astype(jnp.bfloat16), jnp.uint16).astype(jnp.uint32)

    tbl0 = lax.bitcast_convert_type((to16(pf[:, 0]) << 16) | to16(pf[:, 1]),
                                    jnp.int32)
    zb = lax.bitcast_convert_type(pf[:, 2], jnp.uint32)
    tbl1 = lax.bitcast_convert_type(
        (zb & jnp.uint32(0xFFFFFFC0))
        | (elm.reshape(-1).astype(jnp.uint32) * 9),
        jnp.int32)
    kk_tbl = (k[:, None] + k[None, :]).reshape(-1)
    rr_tbl = (radius[:, None] + radius[None, :]).reshape(-1)
    kr_tbl = lax.bitcast_convert_type((to16(kk_tbl) << 16) | to16(rr_tbl),
                                      jnp.int32)

    call = _sc_call(B * A, A, E, nelem)
    partial = call(tbl0, tbl1, kr_tbl, adj.astype(jnp.int32))
    return partial.reshape(_NW, 16, 16).sum(axis=(0, 2))


# speedup vs baseline: 1.0613x; 1.0210x over previous
import functools

import jax
import jax.numpy as jnp
from jax import lax
from jax.experimental import pallas as pl
from jax.experimental.pallas import tpu as pltpu
from jax.experimental.pallas import tpu_sc as plsc

_NW = 32
_TILE = 128
_CT_PER_CHUNK = 15
_CHUNK = _TILE * _CT_PER_CHUNK
_CHUNKS_PER_W = 26
_UNROLL = 6


def _sc_call(num_atoms, atoms_per_b, num_edges, nelem):
    vecs = _CHUNK // 16
    main_tiles = _NW * _CHUNKS_PER_W * _CT_PER_CHUNK
    tail_tiles = num_edges // _TILE - main_tiles
    mesh = plsc.VectorSubcoreMesh(core_axis_name="c", subcore_axis_name="s")

    @functools.partial(
        pl.kernel,
        out_type=jax.ShapeDtypeStruct((_NW, 256), jnp.float32),
        mesh=mesh,
        scratch_types=[
            pltpu.VMEM((num_atoms,), jnp.int32),
            pltpu.VMEM((num_atoms,), jnp.int32),
            pltpu.VMEM((nelem * nelem,), jnp.int32),
            pltpu.VMEM((3, _CHUNK), jnp.int32),
            pltpu.VMEM((3, _CHUNK), jnp.int32),
            pltpu.VMEM((256,), jnp.float32),
            pltpu.SemaphoreType.DMA,
            pltpu.SemaphoreType.DMA,
            pltpu.SemaphoreType.DMA,
        ],
        compiler_params=pltpu.CompilerParams(needs_layout_passes=False),
    )
    def body(tbl0_h, tbl1_h, kr_h, adj_h, out_h,
             tbl0, tbl1, krv, ab0, ab1, acc, semt, sem0, sem1):
        wid = lax.axis_index("s") * 2 + lax.axis_index("c")
        base_w = wid * (_CHUNKS_PER_W * _CHUNK)

        def chunk_src(c):
            return adj_h.at[:, pl.ds(base_w + c * _CHUNK, _CHUNK)]

        pltpu.async_copy(chunk_src(0), ab0, sem0)
        dt0 = pltpu.async_copy(tbl0_h, tbl0, semt)
        dt1 = pltpu.async_copy(tbl1_h, tbl1, semt)
        dt2 = pltpu.async_copy(kr_h, krv, semt)
        for r in range(16):
            acc[pl.ds(r * 16, 16)] = jnp.zeros((16,), jnp.float32)
        lane = lax.iota(jnp.int32, 16)
        mh = jnp.int32(-65536)
        dt0.wait()
        dt1.wait()
        dt2.wait()

        def compute(ab, nvec):
            @plsc.parallel_loop(0, nvec, 1, unroll=_UNROLL)
            def vec_body(v):
                o = v * 16
                nn = ab[0, pl.ds(o, 16)]
                ii = ab[1, pl.ds(o, 16)]
                jj = ab[2, pl.ds(o, 16)]
                gb = nn * atoms_per_b
                gi = gb + ii
                gj = gb + jj
                w0i = plsc.load_gather(tbl0, [gi])
                w1i = plsc.load_gather(tbl1, [gi])
                w0j = plsc.load_gather(tbl0, [gj])
                w1j = plsc.load_gather(tbl1, [gj])
                xi = plsc.bitcast(w0i & mh, jnp.float32)
                yi = plsc.bitcast(w0i << 16, jnp.float32)
                xj = plsc.bitcast(w0j & mh, jnp.float32)
                yj = plsc.bitcast(w0j << 16, jnp.float32)
                zi = plsc.bitcast(w1i & jnp.int32(-64), jnp.float32)
                zj = plsc.bitcast(w1j & jnp.int32(-64), jnp.float32)
                code = (w1i & 56) | (w1j & 7)
                krc = plsc.load_gather(krv, [code])
                kk = plsc.bitcast(krc & mh, jnp.float32)
                rr = plsc.bitcast(krc << 16, jnp.float32)
                dx = xj - xi
                dy = yj - yi
                dz = zj - zi
                sod = dx * dx + dy * dy + dz * dz
                yh = plsc.bitcast(
                    jnp.int32(0x5F3759DF) - (plsc.bitcast(sod, jnp.int32) >> 1),
                    jnp.float32)
                half = 0.5 * sod
                yh = yh * (1.5 - half * yh * yh)
                yh = yh * (1.5 - half * yh * yh)
                dis = sod * yh
                dd = jnp.minimum(dis - rr, jnp.float32(0.0))
                eng = kk * dd * dd
                plsc.addupdate_scatter(acc, [(nn << 4) + lane], eng)

        def pair_body(g, carry):
            c0 = 2 * g
            pltpu.async_copy(chunk_src(c0 + 1), ab1, sem1)
            pltpu.make_async_copy(chunk_src(c0), ab0, sem0).wait()
            compute(ab0, vecs)

            @pl.when(c0 + 2 < _CHUNKS_PER_W)
            def _():
                pltpu.async_copy(chunk_src(c0 + 2), ab0, sem0)

            pltpu.make_async_copy(chunk_src(c0 + 1), ab1, sem1).wait()
            compute(ab1, vecs)
            return carry

        lax.fori_loop(0, _CHUNKS_PER_W // 2, pair_body, 0, unroll=False)

        @pl.when(wid < tail_tiles)
        def _():
            tail_src = adj_h.at[:, pl.ds((main_tiles + wid) * _TILE, _TILE)]
            tail_dst = ab0.at[:, pl.ds(0, _TILE)]
            pltpu.async_copy(tail_src, tail_dst, sem0)
            pltpu.make_async_copy(tail_src, tail_dst, sem0).wait()
            compute(ab0, _TILE // 16)

        pltpu.sync_copy(acc, out_h.at[wid])

    return body


def kernel(pos, elm, adj, k, radius):
    B, A, _ = pos.shape
    E = adj.shape[1]
    nelem = k.shape[0]

    pf = pos.reshape(B * A, 3)

    def to16(v):
        return lax.bitcast_convert_type(
            v.astype(jnp.bfloat16), jnp.uint16).astype(jnp.uint32)

    tbl0 = lax.bitcast_convert_type((to16(pf[:, 0]) << 16) | to16(pf[:, 1]),
                                    jnp.int32)
    zb = lax.bitcast_convert_type(pf[:, 2], jnp.uint32)
    tbl1 = lax.bitcast_convert_type(
        (zb & jnp.uint32(0xFFFFFFC0))
        | (elm.reshape(-1).astype(jnp.uint32) * 9),
        jnp.int32)
    kk_tbl = (k[:, None] + k[None, :]).reshape(-1)
    rr_tbl = (radius[:, None] + radius[None, :]).reshape(-1)
    kr_tbl = lax.bitcast_convert_type((to16(kk_tbl) << 16) | to16(rr_tbl),
                                      jnp.int32)

    call = _sc_call(B * A, A, E, nelem)
    partial = call(tbl0, tbl1, kr_tbl, adj.astype(jnp.int32))
    return partial.reshape(_NW, 16, 16).sum(axis=(0, 2))


# speedup vs baseline: 1.0615x; 1.0002x over previous
import functools

import jax
import jax.numpy as jnp
from jax import lax
from jax.experimental import pallas as pl
from jax.experimental.pallas import tpu as pltpu
from jax.experimental.pallas import tpu_sc as plsc

_NW = 32
_TILE = 128
_CT_PER_CHUNK = 15
_CHUNK = _TILE * _CT_PER_CHUNK
_CHUNKS_PER_W = 26
_UNROLL = 5


def _sc_call(num_atoms, atoms_per_b, num_edges, nelem):
    vecs = _CHUNK // 16
    main_tiles = _NW * _CHUNKS_PER_W * _CT_PER_CHUNK
    tail_tiles = num_edges // _TILE - main_tiles
    mesh = plsc.VectorSubcoreMesh(core_axis_name="c", subcore_axis_name="s")

    @functools.partial(
        pl.kernel,
        out_type=jax.ShapeDtypeStruct((_NW, 256), jnp.float32),
        mesh=mesh,
        scratch_types=[
            pltpu.VMEM((num_atoms,), jnp.int32),
            pltpu.VMEM((num_atoms,), jnp.int32),
            pltpu.VMEM((nelem * nelem,), jnp.int32),
            pltpu.VMEM((3, _CHUNK), jnp.int32),
            pltpu.VMEM((3, _CHUNK), jnp.int32),
            pltpu.VMEM((256,), jnp.float32),
            pltpu.SemaphoreType.DMA,
            pltpu.SemaphoreType.DMA,
            pltpu.SemaphoreType.DMA,
        ],
        compiler_params=pltpu.CompilerParams(needs_layout_passes=False),
    )
    def body(tbl0_h, tbl1_h, kr_h, adj_h, out_h,
             tbl0, tbl1, krv, ab0, ab1, acc, semt, sem0, sem1):
        wid = lax.axis_index("s") * 2 + lax.axis_index("c")
        base_w = wid * (_CHUNKS_PER_W * _CHUNK)

        def chunk_src(c):
            return adj_h.at[:, pl.ds(base_w + c * _CHUNK, _CHUNK)]

        pltpu.async_copy(chunk_src(0), ab0, sem0)
        dt0 = pltpu.async_copy(tbl0_h, tbl0, semt)
        dt1 = pltpu.async_copy(tbl1_h, tbl1, semt)
        dt2 = pltpu.async_copy(kr_h, krv, semt)
        for r in range(16):
            acc[pl.ds(r * 16, 16)] = jnp.zeros((16,), jnp.float32)
        lane = lax.iota(jnp.int32, 16)
        mh = jnp.int32(-65536)
        dt0.wait()
        dt1.wait()
        dt2.wait()

        def compute(ab, nvec):
            @plsc.parallel_loop(0, nvec, 1, unroll=_UNROLL)
            def vec_body(v):
                o = v * 16
                nn = ab[0, pl.ds(o, 16)]
                ii = ab[1, pl.ds(o, 16)]
                jj = ab[2, pl.ds(o, 16)]
                gb = nn * atoms_per_b
                gi = gb + ii
                gj = gb + jj
                w0i = plsc.load_gather(tbl0, [gi])
                w1i = plsc.load_gather(tbl1, [gi])
                w0j = plsc.load_gather(tbl0, [gj])
                w1j = plsc.load_gather(tbl1, [gj])
                xi = plsc.bitcast(w0i & mh, jnp.float32)
                yi = plsc.bitcast(w0i << 16, jnp.float32)
                xj = plsc.bitcast(w0j & mh, jnp.float32)
                yj = plsc.bitcast(w0j << 16, jnp.float32)
                zi = plsc.bitcast(w1i & jnp.int32(-64), jnp.float32)
                zj = plsc.bitcast(w1j & jnp.int32(-64), jnp.float32)
                code = (w1i & 56) | (w1j & 7)
                krc = plsc.load_gather(krv, [code])
                kk = plsc.bitcast(krc & mh, jnp.float32)
                rr = plsc.bitcast(krc << 16, jnp.float32)
                dx = xj - xi
                dy = yj - yi
                dz = zj - zi
                sod = dx * dx + dy * dy + dz * dz
                yh = plsc.bitcast(
                    jnp.int32(0x5F3759DF) - (plsc.bitcast(sod, jnp.int32) >> 1),
                    jnp.float32)
                half = 0.5 * sod
                yh = yh * (1.5 - half * yh * yh)
                yh = yh * (1.5 - half * yh * yh)
                dis = sod * yh
                dd = jnp.minimum(dis - rr, jnp.float32(0.0))
                eng = kk * dd * dd
                plsc.addupdate_scatter(acc, [(nn << 4) + lane], eng)

        def pair_body(g, carry):
            c0 = 2 * g
            pltpu.async_copy(chunk_src(c0 + 1), ab1, sem1)
            pltpu.make_async_copy(chunk_src(c0), ab0, sem0).wait()
            compute(ab0, vecs)

            @pl.when(c0 + 2 < _CHUNKS_PER_W)
            def _():
                pltpu.async_copy(chunk_src(c0 + 2), ab0, sem0)

            pltpu.make_async_copy(chunk_src(c0 + 1), ab1, sem1).wait()
            compute(ab1, vecs)
            return carry

        lax.fori_loop(0, _CHUNKS_PER_W // 2, pair_body, 0, unroll=False)

        @pl.when(wid < tail_tiles)
        def _():
            tail_src = adj_h.at[:, pl.ds((main_tiles + wid) * _TILE, _TILE)]
            tail_dst = ab0.at[:, pl.ds(0, _TILE)]
            pltpu.async_copy(tail_src, tail_dst, sem0)
            pltpu.make_async_copy(tail_src, tail_dst, sem0).wait()
            compute(ab0, _TILE // 16)

        pltpu.sync_copy(acc, out_h.at[wid])

    return body


def kernel(pos, elm, adj, k, radius):
    B, A, _ = pos.shape
    E = adj.shape[1]
    nelem = k.shape[0]

    pf = pos.reshape(B * A, 3)

    def to16(v):
        return lax.bitcast_convert_type(
            v.astype(jnp.bfloat16), jnp.uint16).astype(jnp.uint32)

    tbl0 = lax.bitcast_convert_type((to16(pf[:, 0]) << 16) | to16(pf[:, 1]),
                                    jnp.int32)
    zb = lax.bitcast_convert_type(pf[:, 2], jnp.uint32)
    tbl1 = lax.bitcast_convert_type(
        (zb & jnp.uint32(0xFFFFFFC0))
        | (elm.reshape(-1).astype(jnp.uint32) * 9),
        jnp.int32)
    kk_tbl = (k[:, None] + k[None, :]).reshape(-1)
    rr_tbl = (radius[:, None] + radius[None, :]).reshape(-1)
    kr_tbl = lax.bitcast_convert_type((to16(kk_tbl) << 16) | to16(rr_tbl),
                                      jnp.int32)

    call = _sc_call(B * A, A, E, nelem)
    partial = call(tbl0, tbl1, kr_tbl, adj.astype(jnp.int32))
    return partial.reshape(_NW, 16, 16).sum(axis=(0, 2))


# speedup vs baseline: 1.1070x; 1.0429x over previous
import functools

import jax
import jax.numpy as jnp
from jax import lax
from jax.experimental import pallas as pl
from jax.experimental.pallas import tpu as pltpu
from jax.experimental.pallas import tpu_sc as plsc

_NW = 32
_TILE = 128
_CT_PER_CHUNK = 15
_CHUNK = _TILE * _CT_PER_CHUNK
_CHUNKS_PER_W = 26
_UNROLL = 5


def _sc_call(num_atoms, atoms_per_b, num_edges, nelem):
    vecs = _CHUNK // 16
    main_tiles = _NW * _CHUNKS_PER_W * _CT_PER_CHUNK
    tail_tiles = num_edges // _TILE - main_tiles
    mesh = plsc.VectorSubcoreMesh(core_axis_name="c", subcore_axis_name="s")

    @functools.partial(
        pl.kernel,
        out_type=jax.ShapeDtypeStruct((_NW, 256), jnp.float32),
        mesh=mesh,
        scratch_types=[
            pltpu.VMEM((num_atoms,), jnp.int32),
            pltpu.VMEM((num_atoms,), jnp.int32),
            pltpu.VMEM((nelem * nelem,), jnp.int32),
            pltpu.VMEM((3, _CHUNK), jnp.int32),
            pltpu.VMEM((3, _CHUNK), jnp.int32),
            pltpu.VMEM((256,), jnp.float32),
            pltpu.SemaphoreType.DMA,
            pltpu.SemaphoreType.DMA,
            pltpu.SemaphoreType.DMA,
        ],
        compiler_params=pltpu.CompilerParams(needs_layout_passes=False),
    )
    def body(tbl0_h, tbl1_h, kr_h, adj_h, out_h,
             tbl0, tbl1, krv, ab0, ab1, acc, semt, sem0, sem1):
        wid = lax.axis_index("s") * 2 + lax.axis_index("c")
        base_w = wid * (_CHUNKS_PER_W * _CHUNK)

        def chunk_src(c):
            return adj_h.at[:, pl.ds(base_w + c * _CHUNK, _CHUNK)]

        pltpu.async_copy(chunk_src(0), ab0, sem0)
        dt0 = pltpu.async_copy(tbl0_h, tbl0, semt)
        dt1 = pltpu.async_copy(tbl1_h, tbl1, semt)
        dt2 = pltpu.async_copy(kr_h, krv, semt)
        for r in range(16):
            acc[pl.ds(r * 16, 16)] = jnp.zeros((16,), jnp.float32)
        lane = lax.iota(jnp.int32, 16)
        mh = jnp.int32(-65536)
        dt0.wait()
        dt1.wait()
        dt2.wait()

        def compute(ab, nvec):
            @plsc.parallel_loop(0, nvec, 1, unroll=_UNROLL)
            def vec_body(v):
                o = v * 16
                nn = ab[0, pl.ds(o, 16)]
                ii = ab[1, pl.ds(o, 16)]
                jj = ab[2, pl.ds(o, 16)]
                gb = nn * atoms_per_b
                gi = gb + ii
                gj = gb + jj
                w0i = plsc.load_gather(tbl0, [gi])
                w1i = plsc.load_gather(tbl1, [gi])
                w0j = plsc.load_gather(tbl0, [gj])
                w1j = plsc.load_gather(tbl1, [gj])
                xi = plsc.bitcast(w0i & mh, jnp.float32)
                yi = plsc.bitcast(w0i << 16, jnp.float32)
                xj = plsc.bitcast(w0j & mh, jnp.float32)
                yj = plsc.bitcast(w0j << 16, jnp.float32)
                zi = plsc.bitcast(w1i & jnp.int32(-64), jnp.float32)
                zj = plsc.bitcast(w1j & jnp.int32(-64), jnp.float32)
                code = (w1i & 56) | (w1j & 7)
                krc = plsc.load_gather(krv, [code])
                kk = plsc.bitcast(krc & mh, jnp.float32)
                rr = plsc.bitcast(krc << 16, jnp.float32)
                dx = xj - xi
                dy = yj - yi
                dz = zj - zi
                sod = dx * dx + dy * dy + dz * dz
                yh = plsc.bitcast(
                    jnp.int32(0x5F3759DF) - (plsc.bitcast(sod, jnp.int32) >> 1),
                    jnp.float32)
                half = 0.5 * sod
                yh = yh * (1.5 - half * yh * yh)
                dis = sod * yh
                dd = jnp.minimum(dis - rr, jnp.float32(0.0))
                eng = kk * dd * dd
                plsc.addupdate_scatter(acc, [(nn << 4) + lane], eng)

        def pair_body(g, carry):
            c0 = 2 * g
            pltpu.async_copy(chunk_src(c0 + 1), ab1, sem1)
            pltpu.make_async_copy(chunk_src(c0), ab0, sem0).wait()
            compute(ab0, vecs)

            @pl.when(c0 + 2 < _CHUNKS_PER_W)
            def _():
                pltpu.async_copy(chunk_src(c0 + 2), ab0, sem0)

            pltpu.make_async_copy(chunk_src(c0 + 1), ab1, sem1).wait()
            compute(ab1, vecs)
            return carry

        lax.fori_loop(0, _CHUNKS_PER_W // 2, pair_body, 0, unroll=False)

        @pl.when(wid < tail_tiles)
        def _():
            tail_src = adj_h.at[:, pl.ds((main_tiles + wid) * _TILE, _TILE)]
            tail_dst = ab0.at[:, pl.ds(0, _TILE)]
            pltpu.async_copy(tail_src, tail_dst, sem0)
            pltpu.make_async_copy(tail_src, tail_dst, sem0).wait()
            compute(ab0, _TILE // 16)

        pltpu.sync_copy(acc, out_h.at[wid])

    return body


def kernel(pos, elm, adj, k, radius):
    B, A, _ = pos.shape
    E = adj.shape[1]
    nelem = k.shape[0]

    pf = pos.reshape(B * A, 3)

    def to16(v):
        return lax.bitcast_convert_type(
            v.astype(jnp.bfloat16), jnp.uint16).astype(jnp.uint32)

    tbl0 = lax.bitcast_convert_type((to16(pf[:, 0]) << 16) | to16(pf[:, 1]),
                                    jnp.int32)
    zb = lax.bitcast_convert_type(pf[:, 2], jnp.uint32)
    tbl1 = lax.bitcast_convert_type(
        (zb & jnp.uint32(0xFFFFFFC0))
        | (elm.reshape(-1).astype(jnp.uint32) * 9),
        jnp.int32)
    kk_tbl = (k[:, None] + k[None, :]).reshape(-1)
    rr_tbl = (radius[:, None] + radius[None, :]).reshape(-1)
    kr_tbl = lax.bitcast_convert_type((to16(kk_tbl) << 16) | to16(rr_tbl),
                                      jnp.int32)

    call = _sc_call(B * A, A, E, nelem)
    partial = call(tbl0, tbl1, kr_tbl, adj.astype(jnp.int32))
    return partial.reshape(_NW, 16, 16).sum(axis=(0, 2))
